# baseline (speedup 1.0000x reference)
import functools

import jax
import jax.numpy as jnp
from jax.experimental import pallas as pl
from jax.experimental.pallas import tpu as pltpu

_S = 1024


def _pool_body(seg_ref, x_ref, w1_ref, b1_ref, w2_ref, out_ref, denom_ref,
               *, nb, s):
    i = pl.program_id(0)

    @pl.when(i == 0)
    def _init():
        out_ref[...] = jnp.zeros_like(out_ref)
        denom_ref[...] = jnp.zeros_like(denom_ref)

    xb = x_ref[...]
    h = jnp.tanh(
        jnp.dot(xb, w1_ref[...], preferred_element_type=jnp.float32)
        + b1_ref[...])
    wv = jnp.dot(h, w2_ref[...], preferred_element_type=jnp.float32)
    e = jnp.exp(wv)

    seg = seg_ref[0]
    p = (jax.lax.broadcasted_iota(jnp.int32, (s, seg.shape[-1]), 0)
         == seg).astype(jnp.bfloat16)
    xw = (xb * e).astype(jnp.bfloat16)
    out_ref[...] += jnp.dot(p, xw, preferred_element_type=jnp.float32)
    denom_ref[...] += jnp.dot(p, e.astype(jnp.bfloat16),
                              preferred_element_type=jnp.float32)

    @pl.when(i == nb - 1)
    def _finish():
        out_ref[...] = out_ref[...] / (denom_ref[...] + 1e-16)


def _pooling_call(x, seg3, w1, b1r, w2, *, bn, nb, s, d, interpret=False):
    return pl.pallas_call(
        functools.partial(_pool_body, nb=nb, s=s),
        grid=(nb,),
        in_specs=[
            pl.BlockSpec((1, 1, bn), lambda i: (i, 0, 0)),
            pl.BlockSpec((bn, d), lambda i: (i, 0)),
            pl.BlockSpec((d, d), lambda i: (0, 0)),
            pl.BlockSpec((1, d), lambda i: (0, 0)),
            pl.BlockSpec((d, 1), lambda i: (0, 0)),
        ],
        out_specs=pl.BlockSpec((s, d), lambda i: (0, 0)),
        out_shape=jax.ShapeDtypeStruct((s, d), jnp.float32),
        scratch_shapes=[pltpu.VMEM((s, 1), jnp.float32)],
        compiler_params=pltpu.CompilerParams(
            dimension_semantics=("arbitrary",)),
        interpret=interpret,
    )(seg3, x, w1, b1r, w2)


def kernel(x, batch, W1, b1, W2, b2):
    n, d = x.shape
    bn = 1600
    nb = n // bn
    seg3 = batch.astype(jnp.int32).reshape(nb, 1, bn)
    return _pooling_call(x, seg3, W1, b1.reshape(1, d), W2,
                         bn=bn, nb=nb, s=_S, d=d)


# speedup vs baseline: 1.9510x; 1.9510x over previous
import functools

import jax
import jax.numpy as jnp
from jax.experimental import pallas as pl
from jax.experimental.pallas import tpu as pltpu

_S = 1024


def _pool_body(seg_ref, x_ref, w1_ref, b1_ref, w2_ref, out_ref, denom_ref,
               *, nb, s, sw):
    i = pl.program_id(0)

    @pl.when(i == 0)
    def _init():
        out_ref[...] = jnp.zeros_like(out_ref)
        denom_ref[...] = jnp.zeros_like(denom_ref)

    xb = x_ref[...]
    h = jnp.tanh(
        jnp.dot(xb, w1_ref[...], preferred_element_type=jnp.float32)
        + b1_ref[...])
    wv = jnp.dot(h, w2_ref[...], preferred_element_type=jnp.float32)
    e = jnp.exp(wv)

    seg = seg_ref[0]
    bn = seg.shape[-1]
    xw = (xb * e).astype(jnp.bfloat16)
    e16 = e.astype(jnp.bfloat16)
    smin = jnp.min(seg)
    smax = jnp.max(seg)
    for j in range(s // sw):
        lo = j * sw

        @pl.when(jnp.logical_and(smin < lo + sw, smax >= lo))
        def _win(lo=lo):
            pj = (jax.lax.broadcasted_iota(jnp.int32, (sw, bn), 0) + lo
                  == seg).astype(jnp.bfloat16)
            out_ref[lo:lo + sw, :] += jnp.dot(
                pj, xw, preferred_element_type=jnp.float32)
            denom_ref[lo:lo + sw, :] += jnp.dot(
                pj, e16, preferred_element_type=jnp.float32)

    @pl.when(i == nb - 1)
    def _finish():
        out_ref[...] = out_ref[...] / (denom_ref[...] + 1e-16)


def _pooling_call(x, seg3, w1, b1r, w2, *, bn, nb, s, d, sw=256,
                  interpret=False):
    return pl.pallas_call(
        functools.partial(_pool_body, nb=nb, s=s, sw=sw),
        grid=(nb,),
        in_specs=[
            pl.BlockSpec((1, 1, bn), lambda i: (i, 0, 0)),
            pl.BlockSpec((bn, d), lambda i: (i, 0)),
            pl.BlockSpec((d, d), lambda i: (0, 0)),
            pl.BlockSpec((1, d), lambda i: (0, 0)),
            pl.BlockSpec((d, 1), lambda i: (0, 0)),
        ],
        out_specs=pl.BlockSpec((s, d), lambda i: (0, 0)),
        out_shape=jax.ShapeDtypeStruct((s, d), jnp.float32),
        scratch_shapes=[pltpu.VMEM((s, 1), jnp.float32)],
        compiler_params=pltpu.CompilerParams(
            dimension_semantics=("arbitrary",)),
        interpret=interpret,
    )(seg3, x, w1, b1r, w2)


def kernel(x, batch, W1, b1, W2, b2):
    n, d = x.shape
    bn = 1600
    nb = n // bn
    seg3 = batch.astype(jnp.int32).reshape(nb, 1, bn)
    return _pooling_call(x, seg3, W1, b1.reshape(1, d), W2,
                         bn=bn, nb=nb, s=_S, d=d)


# speedup vs baseline: 3.0770x; 1.5771x over previous
import functools

import jax
import jax.numpy as jnp
from jax.experimental import pallas as pl
from jax.experimental.pallas import tpu as pltpu

_S = 1024


def _pool_body(seg_ref, x_ref, w1_ref, b1_ref, w2_ref, out_ref, denom_ref,
               *, nb, s, sw):
    i = pl.program_id(0)

    @pl.when(i == 0)
    def _init():
        out_ref[...] = jnp.zeros_like(out_ref)
        denom_ref[...] = jnp.zeros_like(denom_ref)

    xb = x_ref[...]
    h = jnp.tanh(
        jnp.dot(xb, w1_ref[...], preferred_element_type=jnp.float32)
        + b1_ref[...])
    wv = jnp.dot(h, w2_ref[...], preferred_element_type=jnp.float32)
    e = jnp.exp(wv)

    seg = seg_ref[0]
    bn = seg.shape[-1]
    xw = (xb * e).astype(jnp.bfloat16)
    e16 = e.astype(jnp.bfloat16)
    smin = jnp.min(seg)
    smax = jnp.max(seg)
    for j in range(s // sw):
        lo = j * sw

        @pl.when(jnp.logical_and(smin < lo + sw, smax >= lo))
        def _win(lo=lo):
            pj = (jax.lax.broadcasted_iota(jnp.int32, (sw, bn), 0) + lo
                  == seg).astype(jnp.bfloat16)
            out_ref[lo:lo + sw, :] += jnp.dot(
                pj, xw, preferred_element_type=jnp.float32)
            denom_ref[lo:lo + sw, :] += jnp.dot(
                pj, e16, preferred_element_type=jnp.float32)

    @pl.when(i == nb - 1)
    def _finish():
        out_ref[...] = out_ref[...] / (denom_ref[...] + 1e-16)


def _pooling_call(x, seg3, w1, b1r, w2, *, bn, nb, s, d, sw=256,
                  interpret=False):
    return pl.pallas_call(
        functools.partial(_pool_body, nb=nb, s=s, sw=sw),
        grid=(nb,),
        in_specs=[
            pl.BlockSpec((1, 1, bn), lambda i: (i, 0, 0)),
            pl.BlockSpec((bn, d), lambda i: (i, 0)),
            pl.BlockSpec((d, d), lambda i: (0, 0)),
            pl.BlockSpec((1, d), lambda i: (0, 0)),
            pl.BlockSpec((d, 1), lambda i: (0, 0)),
        ],
        out_specs=pl.BlockSpec((s, d), lambda i: (0, 0)),
        out_shape=jax.ShapeDtypeStruct((s, d), jnp.float32),
        scratch_shapes=[pltpu.VMEM((s, 1), jnp.float32)],
        compiler_params=pltpu.CompilerParams(
            dimension_semantics=("arbitrary",)),
        interpret=interpret,
    )(seg3, x, w1, b1r, w2)


def kernel(x, batch, W1, b1, W2, b2):
    n, d = x.shape
    bn = 3200
    nb = n // bn
    seg3 = batch.astype(jnp.int32).reshape(nb, 1, bn)
    return _pooling_call(x, seg3, W1, b1.reshape(1, d), W2,
                         bn=bn, nb=nb, s=_S, d=d, sw=128)


# speedup vs baseline: 3.6198x; 1.1764x over previous
import functools

import jax
import jax.numpy as jnp
from jax.experimental import pallas as pl
from jax.experimental.pallas import tpu as pltpu

_S = 1024


def _pool_body(seg_ref, x_ref, w1_ref, b1_ref, w2_ref, out_ref, denom_ref,
               *, nb, s, sw):
    i = pl.program_id(0)

    @pl.when(i == 0)
    def _init():
        out_ref[...] = jnp.zeros_like(out_ref)
        denom_ref[...] = jnp.zeros_like(denom_ref)

    xb = x_ref[...]
    h = jnp.tanh(
        jnp.dot(xb, w1_ref[...], preferred_element_type=jnp.float32)
        + b1_ref[...])
    wv = jnp.dot(h, w2_ref[...], preferred_element_type=jnp.float32)
    e = jnp.exp(wv)

    seg = seg_ref[0]
    bn = seg.shape[-1]
    xw = (xb * e).astype(jnp.bfloat16)
    e16 = e.astype(jnp.bfloat16)
    smin = jnp.min(seg)
    smax = jnp.max(seg)
    for j in range(s // sw):
        lo = j * sw

        @pl.when(jnp.logical_and(smin < lo + sw, smax >= lo))
        def _win(lo=lo):
            pj = (jax.lax.broadcasted_iota(jnp.int32, (sw, bn), 0) + lo
                  == seg).astype(jnp.bfloat16)
            out_ref[lo:lo + sw, :] += jnp.dot(
                pj, xw, preferred_element_type=jnp.float32)
            denom_ref[lo:lo + sw, :] += jnp.dot(
                pj, e16, preferred_element_type=jnp.float32)

    @pl.when(i == nb - 1)
    def _finish():
        out_ref[...] = out_ref[...] / (denom_ref[...] + 1e-16)


def _pooling_call(x, seg3, w1, b1r, w2, *, bn, nb, s, d, sw=256,
                  interpret=False):
    return pl.pallas_call(
        functools.partial(_pool_body, nb=nb, s=s, sw=sw),
        grid=(nb,),
        in_specs=[
            pl.BlockSpec((1, 1, bn), lambda i: (i, 0, 0)),
            pl.BlockSpec((bn, d), lambda i: (i, 0)),
            pl.BlockSpec((d, d), lambda i: (0, 0)),
            pl.BlockSpec((1, d), lambda i: (0, 0)),
            pl.BlockSpec((d, 1), lambda i: (0, 0)),
        ],
        out_specs=pl.BlockSpec((s, d), lambda i: (0, 0)),
        out_shape=jax.ShapeDtypeStruct((s, d), jnp.float32),
        scratch_shapes=[pltpu.VMEM((s, 1), jnp.float32)],
        compiler_params=pltpu.CompilerParams(
            dimension_semantics=("arbitrary",)),
        interpret=interpret,
    )(seg3, x, w1, b1r, w2)


def kernel(x, batch, W1, b1, W2, b2):
    n, d = x.shape
    bn = 6400
    nb = n // bn
    seg3 = batch.astype(jnp.int32).reshape(nb, 1, bn)
    return _pooling_call(x, seg3, W1, b1.reshape(1, d), W2,
                         bn=bn, nb=nb, s=_S, d=d, sw=128)


# speedup vs baseline: 3.8040x; 1.0509x over previous
import functools

import jax
import jax.numpy as jnp
from jax.experimental import pallas as pl
from jax.experimental.pallas import tpu as pltpu

_S = 1024


def _pool_body(seg_ref, x_ref, w1_ref, b1_ref, w2_ref, out_ref, denom_ref,
               *, nb, s, sw):
    i = pl.program_id(0)

    @pl.when(i == 0)
    def _init():
        out_ref[...] = jnp.zeros_like(out_ref)
        denom_ref[...] = jnp.zeros_like(denom_ref)

    xb = x_ref[...]
    h = jnp.tanh(
        jnp.dot(xb, w1_ref[...], preferred_element_type=jnp.float32)
        + b1_ref[...])
    wv = jnp.dot(h, w2_ref[...], preferred_element_type=jnp.float32)
    e = jnp.exp(wv)

    seg = seg_ref[0]
    bn = seg.shape[-1]
    xw = (xb * e).astype(jnp.bfloat16)
    e16 = e.astype(jnp.bfloat16)
    smin = jnp.min(seg)
    smax = jnp.max(seg)
    for j in range(s // sw):
        lo = j * sw

        @pl.when(jnp.logical_and(smin < lo + sw, smax >= lo))
        def _win(lo=lo):
            pj = (jax.lax.broadcasted_iota(jnp.int32, (sw, bn), 0) + lo
                  == seg).astype(jnp.bfloat16)
            out_ref[lo:lo + sw, :] += jnp.dot(
                pj, xw, preferred_element_type=jnp.float32)
            denom_ref[lo:lo + sw, :] += jnp.dot(
                pj, e16, preferred_element_type=jnp.float32)

    @pl.when(i == nb - 1)
    def _finish():
        out_ref[...] = out_ref[...] / (denom_ref[...] + 1e-16)


def _pooling_call(x, seg3, w1, b1r, w2, *, bn, nb, s, d, sw=256,
                  interpret=False):
    return pl.pallas_call(
        functools.partial(_pool_body, nb=nb, s=s, sw=sw),
        grid=(nb,),
        in_specs=[
            pl.BlockSpec((1, 1, bn), lambda i: (i, 0, 0)),
            pl.BlockSpec((bn, d), lambda i: (i, 0)),
            pl.BlockSpec((d, d), lambda i: (0, 0)),
            pl.BlockSpec((1, d), lambda i: (0, 0)),
            pl.BlockSpec((d, 1), lambda i: (0, 0)),
        ],
        out_specs=pl.BlockSpec((s, d), lambda i: (0, 0)),
        out_shape=jax.ShapeDtypeStruct((s, d), jnp.float32),
        scratch_shapes=[pltpu.VMEM((s, 1), jnp.float32)],
        compiler_params=pltpu.CompilerParams(
            dimension_semantics=("arbitrary",)),
        interpret=interpret,
    )(seg3, x, w1, b1r, w2)


def kernel(x, batch, W1, b1, W2, b2):
    n, d = x.shape
    bn = 12800
    nb = n // bn
    seg3 = batch.astype(jnp.int32).reshape(nb, 1, bn)
    return _pooling_call(x, seg3, W1, b1.reshape(1, d), W2,
                         bn=bn, nb=nb, s=_S, d=d, sw=128)
